# baseline (speedup 1.0000x reference)
import functools

import jax
import jax.numpy as jnp
from jax import lax
from jax.experimental import pallas as pl
from jax.experimental.pallas import tpu as pltpu
from jax.experimental.pallas import tpu_sc as plsc

VOCAB = 1000000
EMBED = 64
BATCH = 16384
CTX = 20

NC, NS = 2, 16
NW = NC * NS
BPW = BATCH // NW
CHUNK = 16
NCHUNK = BPW // CHUNK
NREG = EMBED // 16
IPW = BPW * CTX
IPC = CHUNK * CTX
NGATHER = 4
IPG = IPC // NGATHER
PADW = 128

PAD_BLK = 4000


def _pad_body(u_ref, out_ref):
    out_ref[:, 0:EMBED] = u_ref[...]
    out_ref[:, EMBED:PADW] = jnp.zeros((PAD_BLK, PADW - EMBED), jnp.float32)


def _cbow_body(ctx_hbm, cen_hbm, up_hbm, v_hbm, out_hbm,
               vmidx, vcen, u_buf, c_buf, res_buf, sem):
    wid = lax.axis_index("s") * NC + lax.axis_index("c")

    pltpu.sync_copy(ctx_hbm.at[pl.ds(wid * IPW, IPW)], vmidx)
    pltpu.sync_copy(cen_hbm.at[pl.ds(wid * BPW, BPW)], vcen)

    lane = lax.iota(jnp.int32, 16)
    inv_ctx = jnp.float32(1.0 / CTX)

    def chunk_body(g, carry):
        for k in range(NGATHER):
            pltpu.async_copy(
                up_hbm.at[vmidx.at[pl.ds(g * IPC + k * IPG, IPG)]],
                u_buf.at[pl.ds(k * IPG, IPG)], sem)
        cvec = vcen[pl.ds(g * CHUNK, CHUNK)]
        for r in range(CHUNK):
            pltpu.async_copy(v_hbm.at[cvec[r]], c_buf.at[r], sem)
        pltpu.make_async_copy(up_hbm.at[pl.ds(0, IPC)], u_buf, sem).wait()
        pltpu.make_async_copy(v_hbm.at[pl.ds(0, CHUNK)], c_buf, sem).wait()

        resv = jnp.zeros((16,), jnp.float32)
        for r in range(CHUNK):
            c_regs = [c_buf[r, pl.ds(t * 16, 16)] for t in range(NREG)]
            accs = [None] * NREG
            for j in range(CTX):
                f = r * CTX + j
                for t in range(NREG):
                    prod = u_buf[f, pl.ds(t * 16, 16)] * c_regs[t]
                    accs[t] = prod if accs[t] is None else accs[t] + prod
            p = (accs[0] + accs[1]) + (accs[2] + accs[3])
            s = jnp.sum(p) * inv_ctx
            resv = jnp.where(lane == r, s, resv)
        res_buf[pl.ds(g * CHUNK, CHUNK)] = resv
        return carry

    lax.fori_loop(0, NCHUNK, chunk_body, 0)
    pltpu.sync_copy(res_buf, out_hbm.at[pl.ds(wid * BPW, BPW)])


@jax.jit
def _cbow(ctx_r, cen_r, embed_u, embed_v):
    pad = pl.pallas_call(
        _pad_body,
        grid=(VOCAB // PAD_BLK,),
        in_specs=[pl.BlockSpec((PAD_BLK, EMBED), lambda i: (i, 0))],
        out_specs=pl.BlockSpec((PAD_BLK, PADW), lambda i: (i, 0)),
        out_shape=jax.ShapeDtypeStruct((VOCAB, PADW), jnp.float32),
    )
    up = pad(embed_u)

    mesh = plsc.VectorSubcoreMesh(core_axis_name="c", subcore_axis_name="s",
                                  num_cores=NC, num_subcores=NS)
    f = pl.kernel(
        _cbow_body,
        out_type=jax.ShapeDtypeStruct((BATCH,), jnp.float32),
        mesh=mesh,
        scratch_types=[
            pltpu.VMEM((IPW,), jnp.int32),
            pltpu.VMEM((BPW,), jnp.int32),
            pltpu.VMEM((IPC, PADW), jnp.float32),
            pltpu.VMEM((CHUNK, EMBED), jnp.float32),
            pltpu.VMEM((BPW,), jnp.float32),
            pltpu.SemaphoreType.DMA,
        ],
        compiler_params=pltpu.CompilerParams(needs_layout_passes=False,
                                             use_tc_tiling_on_sc=True),
    )
    return f(ctx_r, cen_r, up, embed_v)


def kernel(contexts, center, embed_u, embed_v):
    ctx_r = jnp.asarray(contexts, jnp.int32).reshape(BATCH * CTX)
    cen_r = jnp.asarray(center, jnp.int32).reshape(BATCH)
    out = _cbow(ctx_r, cen_r, embed_u, embed_v)
    return out.reshape(BATCH, 1, 1)


# speedup vs baseline: 1.3364x; 1.3364x over previous
import functools

import jax
import jax.numpy as jnp
from jax import lax
from jax.experimental import pallas as pl
from jax.experimental.pallas import tpu as pltpu
from jax.experimental.pallas import tpu_sc as plsc

VOCAB = 1000000
EMBED = 64
BATCH = 16384
CTX = 20

NC, NS = 2, 16
NW = NC * NS
BPW = BATCH // NW
CHUNK = 16
NCHUNK = BPW // CHUNK
NREG = EMBED // 16
IPW = BPW * CTX
IPC = CHUNK * CTX


def _cbow_body(ctx_hbm, cen_hbm, u_hbm, v_hbm, out_hbm,
               vidx, vcen, u_buf, c_buf, res_buf, sem0, sem1):
    wid = lax.axis_index("s") * NC + lax.axis_index("c")

    pltpu.sync_copy(ctx_hbm.at[pl.ds(wid * IPW, IPW)], vidx)
    pltpu.sync_copy(cen_hbm.at[pl.ds(wid * BPW, BPW)], vcen)

    lane = lax.iota(jnp.int32, 16)
    inv_ctx = jnp.float32(1.0 / CTX)
    sems = (sem0, sem1)

    def issue(c, s):
        cvec = vcen[pl.ds(c * CHUNK, CHUNK)]
        jvecs = [vidx[pl.ds(c * IPC + j * CHUNK, CHUNK)]
                 for j in range(CTX)]
        for r in range(CHUNK):
            for j in range(CTX):
                pltpu.async_copy(
                    u_hbm.at[jvecs[j][r]], u_buf.at[s, r * CTX + j], sems[s])
            pltpu.async_copy(v_hbm.at[cvec[r]], c_buf.at[s, r], sems[s])

    def drain(s):
        pltpu.make_async_copy(
            u_hbm.at[pl.ds(0, IPC)], u_buf.at[s], sems[s]).wait()
        pltpu.make_async_copy(
            v_hbm.at[pl.ds(0, CHUNK)], c_buf.at[s], sems[s]).wait()

    def compute(c, s):
        resv = jnp.zeros((16,), jnp.float32)
        for r in range(CHUNK):
            c_regs = [c_buf[s, r, pl.ds(t * 16, 16)] for t in range(NREG)]
            accs = [None] * NREG
            for j in range(CTX):
                f = r * CTX + j
                for t in range(NREG):
                    prod = u_buf[s, f, pl.ds(t * 16, 16)] * c_regs[t]
                    accs[t] = prod if accs[t] is None else accs[t] + prod
            p = (accs[0] + accs[1]) + (accs[2] + accs[3])
            res = jnp.sum(p) * inv_ctx
            resv = jnp.where(lane == r, res, resv)
        res_buf[pl.ds(c * CHUNK, CHUNK)] = resv

    issue(0, 0)

    def pair_body(h, carry):
        issue(2 * h + 1, 1)
        drain(0)
        compute(2 * h, 0)
        issue(lax.rem(2 * h + 2, NCHUNK), 0)
        drain(1)
        compute(2 * h + 1, 1)
        return carry

    lax.fori_loop(0, NCHUNK // 2, pair_body, 0)
    drain(0)
    pltpu.sync_copy(res_buf, out_hbm.at[pl.ds(wid * BPW, BPW)])


@jax.jit
def _cbow(ctx_r, cen_r, embed_u, embed_v):
    mesh = plsc.VectorSubcoreMesh(core_axis_name="c", subcore_axis_name="s",
                                  num_cores=NC, num_subcores=NS)
    f = pl.kernel(
        _cbow_body,
        out_type=jax.ShapeDtypeStruct((BATCH,), jnp.float32),
        mesh=mesh,
        scratch_types=[
            pltpu.VMEM((IPW,), jnp.int32),
            pltpu.VMEM((BPW,), jnp.int32),
            pltpu.VMEM((2, IPC, EMBED), jnp.float32),
            pltpu.VMEM((2, CHUNK, EMBED), jnp.float32),
            pltpu.VMEM((BPW,), jnp.float32),
            pltpu.SemaphoreType.DMA,
            pltpu.SemaphoreType.DMA,
        ],
        compiler_params=pltpu.CompilerParams(needs_layout_passes=False,
                                             use_tc_tiling_on_sc=True),
    )
    return f(ctx_r, cen_r, embed_u, embed_v)


def kernel(contexts, center, embed_u, embed_v):
    ctx_r = jnp.asarray(contexts, jnp.int32).reshape(
        NW, NCHUNK, CHUNK, CTX).transpose(0, 1, 3, 2).reshape(BATCH * CTX)
    cen_r = jnp.asarray(center, jnp.int32).reshape(BATCH)
    out = _cbow(ctx_r, cen_r, embed_u, embed_v)
    return out.reshape(BATCH, 1, 1)
